# Initial kernel scaffold; baseline (speedup 1.0000x reference)
import jax
import jax.numpy as jnp
from jax.experimental import pallas as pl


def kernel(h_V, h_E, E_idx, mask_V, mask_attend, params):
    raise NotImplementedError("write your pallas kernel here")


# speedup vs baseline: 5.0341x; 5.0341x over previous
import functools

import jax
import jax.numpy as jnp
from jax import lax
from jax.experimental import pallas as pl
from jax.experimental.pallas import tpu as pltpu
from jax.experimental.pallas import tpu_sc as plsc

F32 = jnp.float32
_RSQRT2 = 0.7071067811865476


def _gelu(x):
    return 0.5 * x * (1.0 + lax.erf(x * _RSQRT2))


def _layernorm(x, g, b):
    mu = jnp.mean(x, axis=-1, keepdims=True)
    d = x - mu
    var = jnp.mean(d * d, axis=-1, keepdims=True)
    return d * lax.rsqrt(var + 1e-5) * g + b


@functools.lru_cache(maxsize=None)
def _make_gather(n_rows, d, chunk):
    info = plsc.get_sparse_core_info()
    nw = info.num_cores * info.num_subcores
    per_w = n_rows // nw
    n_chunks = per_w // chunk
    mesh = plsc.VectorSubcoreMesh(core_axis_name="c", subcore_axis_name="s")

    @functools.partial(
        pl.kernel,
        mesh=mesh,
        out_type=jax.ShapeDtypeStruct((n_rows, d), F32),
        scratch_types=[
            pltpu.VMEM((chunk,), jnp.int32),
            pltpu.VMEM((chunk, d), F32),
            pltpu.SemaphoreType.DMA,
        ],
    )
    def gather_k(table_hbm, idx_hbm, out_hbm, idx_v, rows_v, sem):
        wid = lax.axis_index("s") * info.num_cores + lax.axis_index("c")
        base = wid * per_w

        def body(i, carry):
            off = base + i * chunk
            pltpu.sync_copy(idx_hbm.at[pl.ds(off, chunk)], idx_v)
            pltpu.async_copy(table_hbm.at[idx_v], rows_v, sem).wait()
            pltpu.sync_copy(rows_v, out_hbm.at[pl.ds(off, chunk)])
            return carry

        lax.fori_loop(0, n_chunks, body, 0)

    return gather_k


def _proj_body(hv_ref, w_ref, out_ref):
    out_ref[...] = jnp.dot(hv_ref[...], w_ref[...], preferred_element_type=F32)


def _round1_body(tl, k, h,
                 hv_ref, he_ref, g_ref, ma_ref, mv_ref,
                 w1a_ref, w1b_ref, w2_ref, w3_ref, win_ref, wout_ref, w11c_ref,
                 b1_ref, b2_ref, b3_ref, bin_ref, bout_ref,
                 ln1g_ref, ln1b_ref, ln2g_ref, ln2b_ref,
                 hv2_ref, p2_ref):
    hv = hv_ref[...]
    a = jnp.dot(hv, w1a_ref[...], preferred_element_type=F32)
    a_rep = jnp.broadcast_to(a[:, None, :], (tl, k, h)).reshape(tl * k, h)
    x = (jnp.dot(he_ref[...], w1b_ref[...], preferred_element_type=F32)
         + g_ref[...] + a_rep + b1_ref[...])
    h1 = _gelu(x)
    h2 = _gelu(jnp.dot(h1, w2_ref[...], preferred_element_type=F32) + b2_ref[...])
    m = jnp.dot(h2, w3_ref[...], preferred_element_type=F32) + b3_ref[...]
    m3 = m.reshape(tl, k, h) * ma_ref[...][:, :, None]
    dh = jnp.sum(m3, axis=1) * (1.0 / 30.0)
    v = _layernorm(hv + dh, ln1g_ref[...], ln1b_ref[...])
    f = jnp.dot(_gelu(jnp.dot(v, win_ref[...], preferred_element_type=F32)
                      + bin_ref[...]),
                wout_ref[...], preferred_element_type=F32) + bout_ref[...]
    v2 = _layernorm(v + f, ln2g_ref[...], ln2b_ref[...]) * mv_ref[...]
    hv2_ref[...] = v2
    p2_ref[...] = jnp.dot(v2, w11c_ref[...], preferred_element_type=F32)


def _round2_body(tl, k, h,
                 hv_ref, he_ref, g_ref,
                 wa_ref, wb_ref, w12_ref, w13_ref,
                 b11_ref, b12_ref, b13_ref, ln3g_ref, ln3b_ref,
                 out_ref):
    a = jnp.dot(hv_ref[...], wa_ref[...], preferred_element_type=F32)
    a_rep = jnp.broadcast_to(a[:, None, :], (tl, k, h)).reshape(tl * k, h)
    he = he_ref[...]
    x = (jnp.dot(he, wb_ref[...], preferred_element_type=F32)
         + g_ref[...] + a_rep + b11_ref[...])
    h1 = _gelu(x)
    h2 = _gelu(jnp.dot(h1, w12_ref[...], preferred_element_type=F32) + b12_ref[...])
    m = jnp.dot(h2, w13_ref[...], preferred_element_type=F32) + b13_ref[...]
    out_ref[...] = _layernorm(he + m, ln3g_ref[...], ln3b_ref[...])


def _full(shape):
    return pl.BlockSpec(shape, lambda i: (0,) * len(shape))


def kernel(h_V, h_E, E_idx, mask_V, mask_attend, params):
    p = params
    _, L, K, H = h_E.shape
    FF = p['Win'].shape[1]
    TL = 64
    EDGE = TL * K

    hv = h_V.reshape(L, H)
    he = h_E.reshape(L * K, H)
    idx = E_idx.reshape(L * K).astype(jnp.int32)
    ma = mask_attend.reshape(L, K)
    mv = mask_V.reshape(L, 1)

    W1 = p['W1']
    w1a, w1b, w1c = W1[:H], W1[H:2 * H], W1[2 * H:]
    W11 = p['W11']
    w11a, w11b, w11c = W11[:H], W11[H:2 * H], W11[2 * H:]
    r1 = lambda a: a.reshape(1, -1)

    p1 = pl.pallas_call(
        _proj_body,
        out_shape=jax.ShapeDtypeStruct((L, H), F32),
    )(hv, w1c)

    gather = _make_gather(L * K, H, 128)
    g1 = gather(p1, idx)

    grid = (L // TL,)
    edge_spec = pl.BlockSpec((EDGE, H), lambda i: (i, 0))
    node_spec = pl.BlockSpec((TL, H), lambda i: (i, 0))
    hv2, p2 = pl.pallas_call(
        functools.partial(_round1_body, TL, K, H),
        grid=grid,
        in_specs=[
            node_spec, edge_spec, edge_spec,
            pl.BlockSpec((TL, K), lambda i: (i, 0)),
            pl.BlockSpec((TL, 1), lambda i: (i, 0)),
            _full((H, H)), _full((H, H)), _full((H, H)), _full((H, H)),
            _full((H, FF)), _full((FF, H)), _full((H, H)),
            _full((1, H)), _full((1, H)), _full((1, H)),
            _full((1, FF)), _full((1, H)),
            _full((1, H)), _full((1, H)), _full((1, H)), _full((1, H)),
        ],
        out_specs=[node_spec, node_spec],
        out_shape=[jax.ShapeDtypeStruct((L, H), F32),
                   jax.ShapeDtypeStruct((L, H), F32)],
        compiler_params=pltpu.CompilerParams(
            dimension_semantics=("arbitrary",)),
    )(hv, he, g1, ma, mv,
      w1a, w1b, p['W2'], p['W3'], p['Win'], p['Wout'], w11c,
      r1(p['b1']), r1(p['b2']), r1(p['b3']), r1(p['bin']), r1(p['bout']),
      r1(p['ln1_g']), r1(p['ln1_b']), r1(p['ln2_g']), r1(p['ln2_b']))

    g2 = gather(p2, idx)

    he_out = pl.pallas_call(
        functools.partial(_round2_body, TL, K, H),
        grid=grid,
        in_specs=[
            node_spec, edge_spec, edge_spec,
            _full((H, H)), _full((H, H)), _full((H, H)), _full((H, H)),
            _full((1, H)), _full((1, H)), _full((1, H)),
            _full((1, H)), _full((1, H)),
        ],
        out_specs=edge_spec,
        out_shape=jax.ShapeDtypeStruct((L * K, H), F32),
        compiler_params=pltpu.CompilerParams(
            dimension_semantics=("arbitrary",)),
    )(hv2, he, g2,
      w11a, w11b, p['W12'], p['W13'],
      r1(p['b11']), r1(p['b12']), r1(p['b13']),
      r1(p['ln3_g']), r1(p['ln3_b']))

    return (hv2.reshape(1, L, H), he_out.reshape(1, L, K, H))


# speedup vs baseline: 6.4279x; 1.2769x over previous
import functools

import jax
import jax.numpy as jnp
from jax import lax
from jax.experimental import pallas as pl
from jax.experimental.pallas import tpu as pltpu
from jax.experimental.pallas import tpu_sc as plsc

F32 = jnp.float32
_RSQRT2 = 0.7071067811865476


def _gelu(x):
    return 0.5 * x * (1.0 + lax.erf(x * _RSQRT2))


def _layernorm(x, g, b):
    mu = jnp.mean(x, axis=-1, keepdims=True)
    d = x - mu
    var = jnp.mean(d * d, axis=-1, keepdims=True)
    return d * lax.rsqrt(var + 1e-5) * g + b


@functools.lru_cache(maxsize=None)
def _make_gather(n_rows, d, chunk=128, sup=3):
    info = plsc.get_sparse_core_info()
    nw = info.num_cores * info.num_subcores
    per_w = n_rows // nw
    sup_rows = sup * chunk
    n_sup = per_w // sup_rows
    n2 = n_sup // 2
    assert per_w == n_sup * sup_rows and n_sup % 2 == 0
    mesh = plsc.VectorSubcoreMesh(core_axis_name="c", subcore_axis_name="s")

    @functools.partial(
        pl.kernel,
        mesh=mesh,
        out_type=jax.ShapeDtypeStruct((n_rows, d), F32),
        scratch_types=[
            pltpu.VMEM((per_w,), jnp.int32),
            pltpu.VMEM((sup_rows, d), F32),
            pltpu.VMEM((sup_rows, d), F32),
            pltpu.SemaphoreType.DMA,
            pltpu.SemaphoreType.DMA,
            pltpu.SemaphoreType.DMA,
            pltpu.SemaphoreType.DMA,
        ],
    )
    def gather_k(table_hbm, idx_hbm, out_hbm, idx_v, r0, r1, g0, g1, s0, s1):
        wid = lax.axis_index("s") * info.num_cores + lax.axis_index("c")
        base = wid * per_w
        pltpu.sync_copy(idx_hbm.at[pl.ds(base, per_w)], idx_v)

        def issue_gather(sup_i, buf, sem):
            for c in range(sup):
                off = sup_i * sup_rows + c * chunk
                pltpu.async_copy(
                    table_hbm.at[idx_v.at[pl.ds(off, chunk)]],
                    buf.at[pl.ds(c * chunk, chunk)], sem)

        def drain_gather(buf, sem):
            pltpu.make_async_copy(
                out_hbm.at[pl.ds(base, sup_rows)], buf, sem).wait()

        def issue_store(sup_i, buf, sem):
            pltpu.async_copy(
                buf, out_hbm.at[pl.ds(base + sup_i * sup_rows, sup_rows)], sem)

        def drain_store(buf, sem):
            pltpu.make_async_copy(
                buf, out_hbm.at[pl.ds(base, sup_rows)], sem).wait()

        issue_gather(0, r0, g0)
        issue_gather(1, r1, g1)

        def body(j, carry):
            a = 2 * j
            drain_gather(r0, g0)
            issue_store(a, r0, s0)

            @pl.when(j < n2 - 1)
            def _():
                drain_store(r0, s0)
                issue_gather(a + 2, r0, g0)

            drain_gather(r1, g1)
            issue_store(a + 1, r1, s1)

            @pl.when(j < n2 - 1)
            def _():
                drain_store(r1, s1)
                issue_gather(a + 3, r1, g1)

            return carry

        lax.fori_loop(0, n2, body, 0)
        drain_store(r0, s0)
        drain_store(r1, s1)

    return gather_k


def _proj_body(hv_ref, w_ref, out_ref):
    out_ref[...] = jnp.dot(hv_ref[...], w_ref[...], preferred_element_type=F32)


def _round1_body(tl, k, h,
                 hv_ref, he_ref, g_ref, ma_ref, mv_ref,
                 w1a_ref, w1b_ref, w2_ref, w3_ref, win_ref, wout_ref, w11c_ref,
                 b1_ref, b2_ref, b3_ref, bin_ref, bout_ref,
                 ln1g_ref, ln1b_ref, ln2g_ref, ln2b_ref,
                 hv2_ref, p2_ref):
    hv = hv_ref[...]
    a = jnp.dot(hv, w1a_ref[...], preferred_element_type=F32)
    a_rep = jnp.broadcast_to(a[:, None, :], (tl, k, h)).reshape(tl * k, h)
    x = (jnp.dot(he_ref[...], w1b_ref[...], preferred_element_type=F32)
         + g_ref[...] + a_rep + b1_ref[...])
    h1 = _gelu(x)
    h2 = _gelu(jnp.dot(h1, w2_ref[...], preferred_element_type=F32) + b2_ref[...])
    m = jnp.dot(h2, w3_ref[...], preferred_element_type=F32) + b3_ref[...]
    m3 = m.reshape(tl, k, h) * ma_ref[...][:, :, None]
    dh = jnp.sum(m3, axis=1) * (1.0 / 30.0)
    v = _layernorm(hv + dh, ln1g_ref[...], ln1b_ref[...])
    f = jnp.dot(_gelu(jnp.dot(v, win_ref[...], preferred_element_type=F32)
                      + bin_ref[...]),
                wout_ref[...], preferred_element_type=F32) + bout_ref[...]
    v2 = _layernorm(v + f, ln2g_ref[...], ln2b_ref[...]) * mv_ref[...]
    hv2_ref[...] = v2
    p2_ref[...] = jnp.dot(v2, w11c_ref[...], preferred_element_type=F32)


def _round2_body(tl, k, h,
                 hv_ref, he_ref, g_ref,
                 wa_ref, wb_ref, w12_ref, w13_ref,
                 b11_ref, b12_ref, b13_ref, ln3g_ref, ln3b_ref,
                 out_ref):
    a = jnp.dot(hv_ref[...], wa_ref[...], preferred_element_type=F32)
    a_rep = jnp.broadcast_to(a[:, None, :], (tl, k, h)).reshape(tl * k, h)
    he = he_ref[...]
    x = (jnp.dot(he, wb_ref[...], preferred_element_type=F32)
         + g_ref[...] + a_rep + b11_ref[...])
    h1 = _gelu(x)
    h2 = _gelu(jnp.dot(h1, w12_ref[...], preferred_element_type=F32) + b12_ref[...])
    m = jnp.dot(h2, w13_ref[...], preferred_element_type=F32) + b13_ref[...]
    out_ref[...] = _layernorm(he + m, ln3g_ref[...], ln3b_ref[...])


def _full(shape):
    return pl.BlockSpec(shape, lambda i: (0,) * len(shape))


def kernel(h_V, h_E, E_idx, mask_V, mask_attend, params):
    p = params
    _, L, K, H = h_E.shape
    FF = p['Win'].shape[1]
    TL = 128
    EDGE = TL * K

    hv = h_V.reshape(L, H)
    he = h_E.reshape(L * K, H)
    idx = E_idx.reshape(L * K).astype(jnp.int32)
    ma = mask_attend.reshape(L, K)
    mv = mask_V.reshape(L, 1)

    W1 = p['W1']
    w1a, w1b, w1c = W1[:H], W1[H:2 * H], W1[2 * H:]
    W11 = p['W11']
    w11a, w11b, w11c = W11[:H], W11[H:2 * H], W11[2 * H:]
    r1 = lambda a: a.reshape(1, -1)

    p1 = pl.pallas_call(
        _proj_body,
        out_shape=jax.ShapeDtypeStruct((L, H), F32),
    )(hv, w1c)

    gather = _make_gather(L * K, H)
    g1 = gather(p1, idx)

    grid = (L // TL,)
    edge_spec = pl.BlockSpec((EDGE, H), lambda i: (i, 0))
    node_spec = pl.BlockSpec((TL, H), lambda i: (i, 0))
    hv2, p2 = pl.pallas_call(
        functools.partial(_round1_body, TL, K, H),
        grid=grid,
        in_specs=[
            node_spec, edge_spec, edge_spec,
            pl.BlockSpec((TL, K), lambda i: (i, 0)),
            pl.BlockSpec((TL, 1), lambda i: (i, 0)),
            _full((H, H)), _full((H, H)), _full((H, H)), _full((H, H)),
            _full((H, FF)), _full((FF, H)), _full((H, H)),
            _full((1, H)), _full((1, H)), _full((1, H)),
            _full((1, FF)), _full((1, H)),
            _full((1, H)), _full((1, H)), _full((1, H)), _full((1, H)),
        ],
        out_specs=[node_spec, node_spec],
        out_shape=[jax.ShapeDtypeStruct((L, H), F32),
                   jax.ShapeDtypeStruct((L, H), F32)],
        compiler_params=pltpu.CompilerParams(
            dimension_semantics=("arbitrary",)),
    )(hv, he, g1, ma, mv,
      w1a, w1b, p['W2'], p['W3'], p['Win'], p['Wout'], w11c,
      r1(p['b1']), r1(p['b2']), r1(p['b3']), r1(p['bin']), r1(p['bout']),
      r1(p['ln1_g']), r1(p['ln1_b']), r1(p['ln2_g']), r1(p['ln2_b']))

    g2 = gather(p2, idx)

    he_out = pl.pallas_call(
        functools.partial(_round2_body, TL, K, H),
        grid=grid,
        in_specs=[
            node_spec, edge_spec, edge_spec,
            _full((H, H)), _full((H, H)), _full((H, H)), _full((H, H)),
            _full((1, H)), _full((1, H)), _full((1, H)),
            _full((1, H)), _full((1, H)),
        ],
        out_specs=edge_spec,
        out_shape=jax.ShapeDtypeStruct((L * K, H), F32),
        compiler_params=pltpu.CompilerParams(
            dimension_semantics=("arbitrary",)),
    )(hv2, he, g2,
      w11a, w11b, p['W12'], p['W13'],
      r1(p['b11']), r1(p['b12']), r1(p['b13']),
      r1(p['ln3_g']), r1(p['ln3_b']))

    return (hv2.reshape(1, L, H), he_out.reshape(1, L, K, H))


# speedup vs baseline: 6.5592x; 1.0204x over previous
import functools

import jax
import jax.numpy as jnp
from jax import lax
from jax.experimental import pallas as pl
from jax.experimental.pallas import tpu as pltpu
from jax.experimental.pallas import tpu_sc as plsc

F32 = jnp.float32


def _egelu(y):
    return y + y * lax.erf(y)


def _layernorm(x, g, b):
    mu = jnp.mean(x, axis=-1, keepdims=True)
    d = x - mu
    var = jnp.mean(d * d, axis=-1, keepdims=True)
    return d * lax.rsqrt(var + 1e-5) * g + b


@functools.lru_cache(maxsize=None)
def _make_gather(n_rows, d, chunk=128, sup=2, nbuf=3):
    info = plsc.get_sparse_core_info()
    nw = info.num_cores * info.num_subcores
    per_w = n_rows // nw
    sup_rows = sup * chunk
    n_sup = per_w // sup_rows
    nj = n_sup // nbuf
    assert per_w == n_sup * sup_rows and n_sup % nbuf == 0
    mesh = plsc.VectorSubcoreMesh(core_axis_name="c", subcore_axis_name="s")

    @functools.partial(
        pl.kernel,
        mesh=mesh,
        out_type=jax.ShapeDtypeStruct((n_rows, d), F32),
        scratch_types=[
            pltpu.VMEM((per_w,), jnp.int32),
        ] + [pltpu.VMEM((sup_rows, d), F32)] * nbuf
          + [pltpu.SemaphoreType.DMA] * (2 * nbuf),
    )
    def gather_k(table_hbm, idx_hbm, out_hbm, idx_v, *bufsem):
        bufs = bufsem[:nbuf]
        gsems = bufsem[nbuf:2 * nbuf]
        ssems = bufsem[2 * nbuf:]
        wid = lax.axis_index("s") * info.num_cores + lax.axis_index("c")
        base = wid * per_w
        pltpu.sync_copy(idx_hbm.at[pl.ds(base, per_w)], idx_v)

        def issue_gather(sup_i, buf, sem):
            for c in range(sup):
                off = sup_i * sup_rows + c * chunk
                pltpu.async_copy(
                    table_hbm.at[idx_v.at[pl.ds(off, chunk)]],
                    buf.at[pl.ds(c * chunk, chunk)], sem)

        def drain_gather(buf, sem):
            pltpu.make_async_copy(
                out_hbm.at[pl.ds(base, sup_rows)], buf, sem).wait()

        def issue_store(sup_i, buf, sem):
            pltpu.async_copy(
                buf, out_hbm.at[pl.ds(base + sup_i * sup_rows, sup_rows)], sem)

        def drain_store(buf, sem):
            pltpu.make_async_copy(
                buf, out_hbm.at[pl.ds(base, sup_rows)], sem).wait()

        for b in range(nbuf):
            issue_gather(b, bufs[b], gsems[b])

        def body(j, carry):
            for b in range(nbuf):
                i = nbuf * j + b
                drain_gather(bufs[b], gsems[b])
                issue_store(i, bufs[b], ssems[b])

                @pl.when(j < nj - 1)
                def _():
                    drain_store(bufs[b], ssems[b])
                    issue_gather(i + nbuf, bufs[b], gsems[b])

            return carry

        lax.fori_loop(0, nj, body, 0)
        for b in range(nbuf):
            drain_store(bufs[b], ssems[b])

    return gather_k


def _proj_body(hv_ref, w_ref, out_ref):
    out_ref[...] = jnp.dot(hv_ref[...], w_ref[...], preferred_element_type=F32)


def _round1_body(tl, k, h,
                 hv_ref, he_ref, g_ref, ma_ref, mv_ref,
                 w1a_ref, w1b_ref, w2_ref, w3_ref, win_ref, wout_ref, w11c_ref,
                 b1_ref, b2_ref, b3_ref, bin_ref, bout_ref,
                 ln1g_ref, ln1b_ref, ln2g_ref, ln2b_ref,
                 hv2_ref, p2_ref):
    hv = hv_ref[...]
    a = jnp.dot(hv, w1a_ref[...], preferred_element_type=F32) + b1_ref[...]
    a_rep = jnp.broadcast_to(a[:, None, :], (tl, k, h)).reshape(tl * k, h)
    y1 = (jnp.dot(he_ref[...], w1b_ref[...], preferred_element_type=F32)
          + a_rep + g_ref[...])
    u1 = _egelu(y1)
    u2 = _egelu(jnp.dot(u1, w2_ref[...], preferred_element_type=F32)
                + b2_ref[...])
    m = jnp.dot(u2, w3_ref[...], preferred_element_type=F32) + b3_ref[...]
    m3 = m.reshape(tl, k, h) * ma_ref[...][:, :, None]
    dh = jnp.sum(m3, axis=1)
    v = _layernorm(hv + dh, ln1g_ref[...], ln1b_ref[...])
    f = jnp.dot(_egelu(jnp.dot(v, win_ref[...], preferred_element_type=F32)
                       + bin_ref[...]),
                wout_ref[...], preferred_element_type=F32) + bout_ref[...]
    v2 = _layernorm(v + f, ln2g_ref[...], ln2b_ref[...]) * mv_ref[...]
    hv2_ref[...] = v2
    p2_ref[...] = jnp.dot(v2, w11c_ref[...], preferred_element_type=F32)


def _round2_body(tl, k, h,
                 hv_ref, he_ref, g_ref,
                 wa_ref, wb_ref, w12_ref, w13_ref,
                 b11_ref, b12_ref, b13_ref, ln3g_ref, ln3b_ref,
                 out_ref):
    a = jnp.dot(hv_ref[...], wa_ref[...], preferred_element_type=F32) \
        + b11_ref[...]
    a_rep = jnp.broadcast_to(a[:, None, :], (tl, k, h)).reshape(tl * k, h)
    he = he_ref[...]
    y1 = (jnp.dot(he, wb_ref[...], preferred_element_type=F32)
          + a_rep + g_ref[...])
    u1 = _egelu(y1)
    u2 = _egelu(jnp.dot(u1, w12_ref[...], preferred_element_type=F32)
                + b12_ref[...])
    m = jnp.dot(u2, w13_ref[...], preferred_element_type=F32) + b13_ref[...]
    out_ref[...] = _layernorm(he + m, ln3g_ref[...], ln3b_ref[...])


def _full(shape):
    return pl.BlockSpec(shape, lambda i: (0,) * len(shape))


def kernel(h_V, h_E, E_idx, mask_V, mask_attend, params):
    p = params
    _, L, K, H = h_E.shape
    FF = p['Win'].shape[1]
    TL = 128
    EDGE = TL * K

    hv = h_V.reshape(L, H)
    he = h_E.reshape(L * K, H)
    idx = E_idx.reshape(L * K).astype(jnp.int32)
    ma = mask_attend.reshape(L, K)
    mv = mask_V.reshape(L, 1)

    s = 2.0 ** -0.5
    W1, W11 = p['W1'], p['W11']
    w1a, w1b, w1c = W1[:H] * s, W1[H:2 * H] * s, W1[2 * H:] * s
    w11a, w11b, w11c = W11[:H] * s, W11[H:2 * H] * s, W11[2 * H:] * s
    r1 = lambda a: a.reshape(1, -1)

    p1 = pl.pallas_call(
        _proj_body,
        out_shape=jax.ShapeDtypeStruct((L, H), F32),
    )(hv, w1c)

    gather = _make_gather(L * K, H)
    g1 = gather(p1, idx)

    grid = (L // TL,)
    edge_spec = pl.BlockSpec((EDGE, H), lambda i: (i, 0))
    node_spec = pl.BlockSpec((TL, H), lambda i: (i, 0))
    hv2, p2 = pl.pallas_call(
        functools.partial(_round1_body, TL, K, H),
        grid=grid,
        in_specs=[
            node_spec, edge_spec, edge_spec,
            pl.BlockSpec((TL, K), lambda i: (i, 0)),
            pl.BlockSpec((TL, 1), lambda i: (i, 0)),
            _full((H, H)), _full((H, H)), _full((H, H)), _full((H, H)),
            _full((H, FF)), _full((FF, H)), _full((H, H)),
            _full((1, H)), _full((1, H)), _full((1, H)),
            _full((1, FF)), _full((1, H)),
            _full((1, H)), _full((1, H)), _full((1, H)), _full((1, H)),
        ],
        out_specs=[node_spec, node_spec],
        out_shape=[jax.ShapeDtypeStruct((L, H), F32),
                   jax.ShapeDtypeStruct((L, H), F32)],
        compiler_params=pltpu.CompilerParams(
            dimension_semantics=("arbitrary",)),
    )(hv, he, g1, ma, mv,
      w1a, w1b, p['W2'] * 0.5, p['W3'] * (s / 30.0), p['Win'] * s,
      p['Wout'] * s, w11c,
      r1(p['b1'] * s), r1(p['b2'] * s), r1(p['b3'] / 30.0),
      r1(p['bin'] * s), r1(p['bout']),
      r1(p['ln1_g']), r1(p['ln1_b']), r1(p['ln2_g']), r1(p['ln2_b']))

    g2 = gather(p2, idx)

    he_out = pl.pallas_call(
        functools.partial(_round2_body, TL, K, H),
        grid=grid,
        in_specs=[
            node_spec, edge_spec, edge_spec,
            _full((H, H)), _full((H, H)), _full((H, H)), _full((H, H)),
            _full((1, H)), _full((1, H)), _full((1, H)),
            _full((1, H)), _full((1, H)),
        ],
        out_specs=edge_spec,
        out_shape=jax.ShapeDtypeStruct((L * K, H), F32),
        compiler_params=pltpu.CompilerParams(
            dimension_semantics=("arbitrary",)),
    )(hv2, he, g2,
      w11a, w11b, p['W12'] * 0.5, p['W13'] * s,
      r1(p['b11'] * s), r1(p['b12'] * s), r1(p['b13']),
      r1(p['ln3_g']), r1(p['ln3_b']))

    return (hv2.reshape(1, L, H), he_out.reshape(1, L, K, H))
